# baseline (speedup 1.0000x reference)
import jax
import jax.numpy as jnp
from jax import lax
from jax.experimental import pallas as pl
from jax.experimental.pallas import tpu as pltpu
from jax.experimental.pallas import tpu_sc as plsc

SEQ = 200
DIM = 64
NC = 2
NS = 16
NW = NC * NS
CHUNK = 64
NBUF = 4

_PAD_BLK = 2000


def _pad_tc_body(w_ref, o_ref):
    o_ref[:, 0:DIM] = w_ref[...]
    o_ref[:, DIM:128] = w_ref[...]


def _emb_body(idx_hbm, word_hbm, pos_hbm, out_hbm, idx_v, pos_v, *bufs):
    gbufs = bufs[0:NBUF]
    obufs = bufs[NBUF:2 * NBUF]
    gsems = bufs[2 * NBUF:3 * NBUF]
    wsems = bufs[3 * NBUF:4 * NBUF]

    w = idx_hbm.shape[0] // NW
    g_cnt = w // CHUNK
    outer = g_cnt // NBUF
    wid = lax.axis_index("s") * NC + lax.axis_index("c")
    base = wid * w

    pltpu.sync_copy(idx_hbm.at[pl.ds(base, w)], idx_v)
    pltpu.sync_copy(pos_hbm, pos_v)

    def issue_gather(g, b):
        pltpu.async_copy(
            word_hbm.at[idx_v.at[pl.ds(g * CHUNK, CHUNK)]], gbufs[b], gsems[b]
        )

    for b in range(NBUF):
        issue_gather(b, b)

    @pl.loop(0, outer)
    def outer_loop(gg):
        for b in range(NBUF):
            g = gg * NBUF + b
            pltpu.make_async_copy(
                word_hbm.at[idx_v.at[pl.ds(0, CHUNK)]], gbufs[b], gsems[b]
            ).wait()

            @pl.when(gg > 0)
            def _():
                pltpu.make_async_copy(
                    obufs[b], out_hbm.at[pl.ds(base, CHUNK)], wsems[b]
                ).wait()

            phase = lax.rem(g * CHUNK, SEQ)

            @pl.loop(0, CHUNK, unroll=4)
            def row_loop(r):
                p = phase + r
                p = jnp.where(p >= SEQ, p - SEQ, p)
                for c in range(DIM // 16):
                    sl = pl.ds(c * 16, 16)
                    obufs[b][r, sl] = gbufs[b][r, sl] + pos_v[p, sl]

            pltpu.async_copy(
                obufs[b], out_hbm.at[pl.ds(base + g * CHUNK, CHUNK)], wsems[b]
            )

            @pl.when(gg + 1 < outer)
            def _():
                issue_gather(g + NBUF, b)

    for b in range(NBUF):
        pltpu.make_async_copy(
            obufs[b], out_hbm.at[pl.ds(base, CHUNK)], wsems[b]
        ).wait()


def kernel(inputs, word_table, pos_table):
    b, seq = inputs.shape
    total = b * seq
    idx_flat = inputs.reshape(total).astype(jnp.int32)
    vocab = word_table.shape[0]

    word_pad = pl.pallas_call(
        _pad_tc_body,
        out_shape=jax.ShapeDtypeStruct((vocab, 128), jnp.float32),
        grid=(vocab // _PAD_BLK,),
        in_specs=[pl.BlockSpec((_PAD_BLK, DIM), lambda i: (i, 0))],
        out_specs=pl.BlockSpec((_PAD_BLK, 128), lambda i: (i, 0)),
    )(word_table)

    mesh = plsc.VectorSubcoreMesh(core_axis_name="c", subcore_axis_name="s")

    call = pl.kernel(
        _emb_body,
        out_type=jax.ShapeDtypeStruct((total, DIM), jnp.float32),
        mesh=mesh,
        scratch_types=[
            pltpu.VMEM((total // NW,), jnp.int32),
            pltpu.VMEM((SEQ, DIM), jnp.float32),
        ]
        + [pltpu.VMEM((CHUNK, 128), jnp.float32) for _ in range(NBUF)]
        + [pltpu.VMEM((CHUNK, DIM), jnp.float32) for _ in range(NBUF)]
        + [pltpu.SemaphoreType.DMA for _ in range(2 * NBUF)],
    )
    out = call(idx_flat, word_pad, pos_table)
    return out.reshape(b, seq, DIM)


# speedup vs baseline: 1.0556x; 1.0556x over previous
import jax
import jax.numpy as jnp
from jax import lax
from jax.experimental import pallas as pl
from jax.experimental.pallas import tpu as pltpu
from jax.experimental.pallas import tpu_sc as plsc

SEQ = 200
DIM = 64
NC = 2
NS = 16
NW = NC * NS
CHUNK = 64
NBUF = 4
NOBUF = 2


def _emb_body(idx_hbm, word_hbm, pos_hbm, out_hbm, idx_v, idx2_v, pos_v,
              *bufs):
    gbufs = bufs[0:NBUF]
    obufs = bufs[NBUF:NBUF + NOBUF]
    gsems = bufs[NBUF + NOBUF:2 * NBUF + NOBUF]
    wsems = bufs[2 * NBUF + NOBUF:2 * NBUF + 2 * NOBUF]

    seqs = idx_hbm.shape[0] // NW
    w = seqs * SEQ
    g_cnt = w // CHUNK
    outer = g_cnt // NBUF
    wid = lax.axis_index("s") * NC + lax.axis_index("c")
    base = wid * w

    pltpu.sync_copy(pos_hbm, pos_v)

    half = seqs // 2
    tail16 = SEQ - 16
    lane = lax.iota(jnp.int32, 16)
    for h in range(2):
        pltpu.sync_copy(
            idx_hbm.at[pl.ds(wid * seqs + h * half, half)], idx2_v
        )

        @pl.loop(0, half)
        def repack(r):
            o = (h * half + r) * SEQ
            for c in range(SEQ // 16):
                idx_v[pl.ds(o + c * 16, 16)] = idx2_v[r, pl.ds(c * 16, 16)]
            idx_v[pl.ds(o + tail16, 16)] = idx2_v[r, pl.ds(tail16, 16)]

    def issue_gather(g, b):
        pltpu.async_copy(
            word_hbm.at[idx_v.at[pl.ds(g * CHUNK, CHUNK)]], gbufs[b], gsems[b]
        )

    for b in range(NBUF):
        issue_gather(b, b)

    @pl.loop(0, outer)
    def outer_loop(gg):
        for b in range(NBUF):
            g = gg * NBUF + b
            ob = b % NOBUF
            pltpu.make_async_copy(
                word_hbm.at[idx_v.at[pl.ds(0, CHUNK)]], gbufs[b], gsems[b]
            ).wait()

            if b >= NOBUF:
                pltpu.make_async_copy(
                    obufs[ob], out_hbm.at[pl.ds(base, CHUNK)], wsems[ob]
                ).wait()
            else:
                @pl.when(gg > 0)
                def _():
                    pltpu.make_async_copy(
                        obufs[ob], out_hbm.at[pl.ds(base, CHUNK)], wsems[ob]
                    ).wait()

            phase = lax.rem(g * CHUNK, SEQ)

            @pl.loop(0, CHUNK, unroll=4)
            def row_loop(r):
                p = phase + r
                p = jnp.where(p >= SEQ, p - SEQ, p)
                for c in range(DIM // 16):
                    sl = pl.ds(c * 16, 16)
                    obufs[ob][r, sl] = gbufs[b][r, sl] + pos_v[p, sl]

            pltpu.async_copy(
                obufs[ob], out_hbm.at[pl.ds(base + g * CHUNK, CHUNK)],
                wsems[ob]
            )

            @pl.when(gg + 1 < outer)
            def _():
                issue_gather(g + NBUF, b)

    for ob in range(NOBUF):
        pltpu.make_async_copy(
            obufs[ob], out_hbm.at[pl.ds(base, CHUNK)], wsems[ob]
        ).wait()


def kernel(inputs, word_table, pos_table):
    b, seq = inputs.shape
    total = b * seq
    vocab = word_table.shape[0]
    word_pad = jnp.pad(word_table, ((0, 0), (0, 128 - DIM)))

    mesh = plsc.VectorSubcoreMesh(core_axis_name="c", subcore_axis_name="s")
    call = pl.kernel(
        _emb_body,
        out_type=jax.ShapeDtypeStruct((total, DIM), jnp.float32),
        mesh=mesh,
        scratch_types=[
            pltpu.VMEM((total // NW,), jnp.int32),
            pltpu.VMEM((b // NW // 2, seq), jnp.int32),
            pltpu.VMEM((SEQ, DIM), jnp.float32),
        ]
        + [pltpu.VMEM((CHUNK, 128), jnp.float32) for _ in range(NBUF)]
        + [pltpu.VMEM((CHUNK, DIM), jnp.float32) for _ in range(NOBUF)]
        + [pltpu.SemaphoreType.DMA for _ in range(NBUF + NOBUF)],
    )
    out = call(inputs.astype(jnp.int32), word_pad, pos_table)
    return out.reshape(b, seq, DIM)
